# Initial kernel scaffold; baseline (speedup 1.0000x reference)
import jax
import jax.numpy as jnp
from jax.experimental import pallas as pl


def kernel(x, x_emb, W):
    raise NotImplementedError("write your pallas kernel here")


# speedup vs baseline: 3.0456x; 3.0456x over previous
import numpy as np
import jax
import jax.numpy as jnp
from jax.experimental import pallas as pl
from jax.experimental.pallas import tpu as pltpu

_K = 64
_TOPK_F = np.float32(_K)
_B = 16384
_NF = 100
_E = 16
_BF = 512
_FLAT = _NF * _E
_NFP = 128
_BLK = 512

_INT_MIN = np.int32(-2147483648)
_POS_MASK = np.int32(0x7FFFFFFF)

_H_NP = np.zeros((_FLAT, _NFP), np.float32)
for _j in range(_FLAT):
    _H_NP[_j, _j // _E] = 1.0

_BITS_NP = [np.int32(np.uint32(1 << (31 - _i)).view(np.int32)) for _i in range(32)]


def _gate_kernel(xe_ref, x_ref, h_ref, wt_ref, o_ref):
    xe = xe_ref[...]
    s_small = jnp.dot(xe, h_ref[...], preferred_element_type=jnp.float32,
                      precision=jax.lax.Precision.HIGHEST)
    ls = s_small * np.float32(1.0 / _E)
    s = jnp.dot(ls.astype(jnp.bfloat16), wt_ref[...].astype(jnp.bfloat16),
                preferred_element_type=jnp.float32)
    s = s + np.float32(0.0)
    bits = jax.lax.bitcast_convert_type(s, jnp.int32)
    ks = jnp.where(bits >= 0, bits, bits ^ _POS_MASK)

    prefix = jnp.zeros((_BLK, 1), jnp.int32)
    for bit in _BITS_NP:
        cand = prefix | bit
        candx = cand ^ _INT_MIN
        cnt = jnp.sum((ks >= candx).astype(jnp.float32), axis=1, keepdims=True)
        prefix = jnp.where(cnt >= _TOPK_F, cand, prefix)

    thr = prefix ^ _INT_MIN
    mask = ks >= thr
    o_ref[...] = jnp.where(mask, x_ref[...], np.float32(0.0))


def kernel(x, x_emb, W):
    xe = x_emb.reshape(_B, _FLAT)
    h = jnp.asarray(_H_NP)
    wt = jnp.zeros((_NFP, _BF), jnp.float32).at[:_NF, :].set(W.T)
    grid = (_B // _BLK,)
    return pl.pallas_call(
        _gate_kernel,
        grid=grid,
        in_specs=[
            pl.BlockSpec((_BLK, _FLAT), lambda i: (i, 0)),
            pl.BlockSpec((_BLK, _BF), lambda i: (i, 0)),
            pl.BlockSpec((_FLAT, _NFP), lambda i: (0, 0)),
            pl.BlockSpec((_NFP, _BF), lambda i: (0, 0)),
        ],
        out_specs=pl.BlockSpec((_BLK, _BF), lambda i: (i, 0)),
        out_shape=jax.ShapeDtypeStruct((_B, _BF), jnp.float32),
        compiler_params=pltpu.CompilerParams(
            dimension_semantics=("arbitrary",),
        ),
    )(xe, x, h, wt)


# speedup vs baseline: 3.4831x; 1.1436x over previous
import numpy as np
import jax
import jax.numpy as jnp
from jax.experimental import pallas as pl
from jax.experimental.pallas import tpu as pltpu

_K = 64
_TOPK_F = np.float32(_K)
_B = 16384
_NF = 100
_E = 16
_BF = 512
_FLAT = _NF * _E
_NFP = 128
_BLK = 512

_INT_MIN = np.int32(-2147483648)
_POS_MASK = np.int32(0x7FFFFFFF)

_H_NP = np.zeros((_FLAT, _NFP), np.float32)
for _j in range(_FLAT):
    _H_NP[_j, _j // _E] = 1.0

_BITS_NP = [np.int32(np.uint32(1 << (31 - _i)).view(np.int32)) for _i in range(32)]


def _gate_kernel(xe_ref, x_ref, h_ref, wt_ref, o_ref):
    xe = xe_ref[...]
    h = h_ref[...].astype(jnp.bfloat16)
    a0 = xe.astype(jnp.bfloat16)
    r1 = xe - a0.astype(jnp.float32)
    a1 = r1.astype(jnp.bfloat16)
    a2 = (r1 - a1.astype(jnp.float32)).astype(jnp.bfloat16)
    s_small = (jnp.dot(a0, h, preferred_element_type=jnp.float32)
               + jnp.dot(a1, h, preferred_element_type=jnp.float32)
               + jnp.dot(a2, h, preferred_element_type=jnp.float32))
    ls = s_small * np.float32(1.0 / _E)
    s = jnp.dot(ls.astype(jnp.bfloat16), wt_ref[...].astype(jnp.bfloat16),
                preferred_element_type=jnp.float32)
    s = s + np.float32(0.0)
    bits = jax.lax.bitcast_convert_type(s, jnp.int32)
    ks = jnp.where(bits >= 0, bits, bits ^ _POS_MASK)

    prefix = jnp.zeros((_BLK, 1), jnp.int32)
    for bit in _BITS_NP:
        cand = prefix | bit
        candx = cand ^ _INT_MIN
        cnt = jnp.sum((ks >= candx).astype(jnp.float32), axis=1, keepdims=True)
        prefix = jnp.where(cnt >= _TOPK_F, cand, prefix)

    thr = prefix ^ _INT_MIN
    mask = ks >= thr
    o_ref[...] = jnp.where(mask, x_ref[...], np.float32(0.0))


def kernel(x, x_emb, W):
    xe = x_emb.reshape(_B, _FLAT)
    h = jnp.asarray(_H_NP)
    wt = jnp.zeros((_NFP, _BF), jnp.float32).at[:_NF, :].set(W.T)
    grid = (_B // _BLK,)
    return pl.pallas_call(
        _gate_kernel,
        grid=grid,
        in_specs=[
            pl.BlockSpec((_BLK, _FLAT), lambda i: (i, 0)),
            pl.BlockSpec((_BLK, _BF), lambda i: (i, 0)),
            pl.BlockSpec((_FLAT, _NFP), lambda i: (0, 0)),
            pl.BlockSpec((_NFP, _BF), lambda i: (0, 0)),
        ],
        out_specs=pl.BlockSpec((_BLK, _BF), lambda i: (i, 0)),
        out_shape=jax.ShapeDtypeStruct((_B, _BF), jnp.float32),
        compiler_params=pltpu.CompilerParams(
            dimension_semantics=("arbitrary",),
        ),
    )(xe, x, h, wt)


# speedup vs baseline: 3.7123x; 1.0658x over previous
import numpy as np
import jax
import jax.numpy as jnp
from jax.experimental import pallas as pl
from jax.experimental.pallas import tpu as pltpu

_K = 64
_TOPK_F = np.float32(_K)
_B = 16384
_NF = 100
_E = 16
_BF = 512
_FLAT = _NF * _E
_NFP = 128
_BLK = 512

_INT_MIN = np.int32(-2147483648)
_POS_MASK = np.int32(0x7FFFFFFF)

_H_NP = np.zeros((_FLAT, _NFP), np.float32)
for _j in range(_FLAT):
    _H_NP[_j, _j // _E] = 1.0

_BITS_NP = [np.int32(np.uint32(1 << (31 - _i)).view(np.int32)) for _i in range(32)]


def _gate_kernel(xe_ref, x_ref, h_ref, wt_ref, o_ref):
    xe = xe_ref[...]
    h = h_ref[...].astype(jnp.bfloat16)
    a0 = xe.astype(jnp.bfloat16)
    r1 = xe - a0.astype(jnp.float32)
    a1 = r1.astype(jnp.bfloat16)
    a2 = (r1 - a1.astype(jnp.float32)).astype(jnp.bfloat16)
    s_small = (jnp.dot(a0, h, preferred_element_type=jnp.float32)
               + jnp.dot(a1, h, preferred_element_type=jnp.float32)
               + jnp.dot(a2, h, preferred_element_type=jnp.float32))
    ls = s_small * np.float32(1.0 / _E)
    s = jnp.dot(ls.astype(jnp.bfloat16), wt_ref[...].astype(jnp.bfloat16),
                preferred_element_type=jnp.float32)
    s = s + np.float32(0.0)
    bits = jax.lax.bitcast_convert_type(s, jnp.int32)
    ks = jnp.where(bits >= 0, bits, bits ^ _POS_MASK)
    kb = ks ^ _INT_MIN

    one16 = np.int16(1)
    zero16 = np.int16(0)

    def _rowcount(m):
        v = jnp.where(m, one16, zero16)
        v = v[:, :256] + v[:, 256:]
        v = v[:, :128] + v[:, 128:]
        return jnp.sum(v.astype(jnp.float32), axis=1, keepdims=True)

    hi16 = (jax.lax.shift_right_logical(kb, 16) ^ 32768).astype(jnp.int16)
    p_hi = jnp.zeros((_BLK, 1), jnp.int32)
    for b in range(15, -1, -1):
        cand = p_hi | (1 << b)
        cand16 = (cand ^ 32768).astype(jnp.int16)
        cnt = _rowcount(hi16 >= cand16)
        p_hi = jnp.where(cnt >= _TOPK_F, cand, p_hi)

    p_hi16 = (p_hi ^ 32768).astype(jnp.int16)
    c_gt = _rowcount(hi16 > p_hi16)
    need = _TOPK_F - c_gt

    lo16 = ((kb & np.int32(0xFFFF)) ^ 32768).astype(jnp.int16)
    lo16m = jnp.where(hi16 == p_hi16, lo16, np.int16(-32768))
    p_lo = jnp.zeros((_BLK, 1), jnp.int32)
    for b in range(15, -1, -1):
        cand = p_lo | (1 << b)
        cand16 = (cand ^ 32768).astype(jnp.int16)
        cnt = _rowcount(lo16m >= cand16)
        p_lo = jnp.where(cnt >= need, cand, p_lo)

    thr = ((p_hi << 16) | p_lo) ^ _INT_MIN
    mask = ks >= thr
    o_ref[...] = jnp.where(mask, x_ref[...], np.float32(0.0))


def kernel(x, x_emb, W):
    xe = x_emb.reshape(_B, _FLAT)
    h = jnp.asarray(_H_NP)
    wt = jnp.zeros((_NFP, _BF), jnp.float32).at[:_NF, :].set(W.T)
    grid = (_B // _BLK,)
    return pl.pallas_call(
        _gate_kernel,
        grid=grid,
        in_specs=[
            pl.BlockSpec((_BLK, _FLAT), lambda i: (i, 0)),
            pl.BlockSpec((_BLK, _BF), lambda i: (i, 0)),
            pl.BlockSpec((_FLAT, _NFP), lambda i: (0, 0)),
            pl.BlockSpec((_NFP, _BF), lambda i: (0, 0)),
        ],
        out_specs=pl.BlockSpec((_BLK, _BF), lambda i: (i, 0)),
        out_shape=jax.ShapeDtypeStruct((_B, _BF), jnp.float32),
        compiler_params=pltpu.CompilerParams(
            dimension_semantics=("arbitrary",),
        ),
    )(xe, x, h, wt)


# speedup vs baseline: 8.6956x; 2.3424x over previous
import numpy as np
import jax
import jax.numpy as jnp
from jax.experimental import pallas as pl
from jax.experimental.pallas import tpu as pltpu

_K = 64
_TOPK_F = np.float32(_K)
_B = 16384
_NF = 100
_E = 16
_BF = 512
_FLAT = _NF * _E
_BLK = 512

_INT_MIN = np.int32(-2147483648)
_POS_MASK = np.int32(0x7FFFFFFF)


def _gate_kernel(xe_ref, x_ref, w_ref, o_ref):
    xe = xe_ref[...]
    g = xe.reshape(_NF, _E, _BLK)
    ls = jnp.sum(g, axis=1) * np.float32(1.0 / _E)
    sT = jnp.dot(w_ref[...], ls.astype(jnp.bfloat16),
                 preferred_element_type=jnp.float32)
    sT = sT + np.float32(0.0)
    bits = jax.lax.bitcast_convert_type(sT, jnp.int32)
    ks = jnp.where(bits >= 0, bits, bits ^ _POS_MASK)
    kb = ks ^ _INT_MIN

    one16 = np.int16(1)
    zero16 = np.int16(0)

    def _colcount(m):
        v = jnp.where(m, one16, zero16)
        v = v[:256] + v[256:]
        v = v[:128] + v[128:]
        v = v[:64] + v[64:]
        v = v[:32] + v[32:]
        v = v[:16] + v[16:]
        return jnp.sum(v.astype(jnp.float32), axis=0, keepdims=True)

    hi16 = (jax.lax.shift_right_logical(kb, 16) ^ 32768).astype(jnp.int16)
    p_hi = jnp.zeros((1, _BLK), jnp.int32)
    for b in range(15, -1, -1):
        cand = p_hi | (1 << b)
        cand16 = (cand ^ 32768).astype(jnp.int16)
        cnt = _colcount(hi16 >= cand16)
        p_hi = jnp.where(cnt >= _TOPK_F, cand, p_hi)

    p_hi16 = (p_hi ^ 32768).astype(jnp.int16)
    c_gt = _colcount(hi16 > p_hi16)
    need = _TOPK_F - c_gt

    lo16 = ((kb & np.int32(0xFFFF)) ^ 32768).astype(jnp.int16)
    lo16m = jnp.where(hi16 == p_hi16, lo16, np.int16(-32768))
    p_lo = jnp.zeros((1, _BLK), jnp.int32)
    for b in range(15, -1, -1):
        cand = p_lo | (1 << b)
        cand16 = (cand ^ 32768).astype(jnp.int16)
        cnt = _colcount(lo16m >= cand16)
        p_lo = jnp.where(cnt >= need, cand, p_lo)

    thr = ((p_hi << 16) | p_lo) ^ _INT_MIN
    maskf = jnp.where(ks >= thr, np.float32(1.0), np.float32(0.0))
    o_ref[...] = x_ref[...] * maskf.T


def kernel(x, x_emb, W):
    xe_t = x_emb.transpose(1, 2, 0).reshape(_FLAT, _B)
    wbf = W.astype(jnp.bfloat16)
    grid = (_B // _BLK,)
    return pl.pallas_call(
        _gate_kernel,
        grid=grid,
        in_specs=[
            pl.BlockSpec((_FLAT, _BLK), lambda i: (0, i)),
            pl.BlockSpec((_BLK, _BF), lambda i: (i, 0)),
            pl.BlockSpec((_BF, _NF), lambda i: (0, 0)),
        ],
        out_specs=pl.BlockSpec((_BLK, _BF), lambda i: (i, 0)),
        out_shape=jax.ShapeDtypeStruct((_B, _BF), jnp.float32),
        compiler_params=pltpu.CompilerParams(
            dimension_semantics=("arbitrary",),
        ),
    )(xe_t, x, wbf)


# speedup vs baseline: 9.3434x; 1.0745x over previous
import numpy as np
import jax
import jax.numpy as jnp
from jax.experimental import pallas as pl
from jax.experimental.pallas import tpu as pltpu

_K = 64
_TOPK_F = np.float32(_K)
_B = 16384
_NF = 100
_E = 16
_BF = 512
_FLAT = _NF * _E
_BLK = 1024

_INT_MIN = np.int32(-2147483648)
_POS_MASK = np.int32(0x7FFFFFFF)


def _gate_kernel(xe_ref, x_ref, w_ref, o_ref):
    xe = xe_ref[...]
    g = xe.reshape(_NF, _E, _BLK)
    ls = jnp.sum(g, axis=1) * np.float32(1.0 / _E)
    sT = jnp.dot(w_ref[...], ls.astype(jnp.bfloat16),
                 preferred_element_type=jnp.float32)
    sT = sT + np.float32(0.0)
    bits = jax.lax.bitcast_convert_type(sT, jnp.int32)
    ks = jnp.where(bits >= 0, bits, bits ^ _POS_MASK)
    kb = ks ^ _INT_MIN

    one16 = np.int16(1)
    zero16 = np.int16(0)

    def _colcount(m):
        v = jnp.where(m, one16, zero16)
        v = v[:256] + v[256:]
        v = v[:128] + v[128:]
        v = v[:64] + v[64:]
        v = v[:32] + v[32:]
        v = v[:16] + v[16:]
        return jnp.sum(v.astype(jnp.float32), axis=0, keepdims=True)

    hi16 = (jax.lax.shift_right_logical(kb, 16) ^ 32768).astype(jnp.int16)
    p_hi = jnp.zeros((1, _BLK), jnp.int32)
    for b in range(15, -1, -1):
        cand = p_hi | (1 << b)
        cand16 = (cand ^ 32768).astype(jnp.int16)
        cnt = _colcount(hi16 >= cand16)
        p_hi = jnp.where(cnt >= _TOPK_F, cand, p_hi)

    p_hi16 = (p_hi ^ 32768).astype(jnp.int16)
    c_gt = _colcount(hi16 > p_hi16)
    need = _TOPK_F - c_gt

    lo16 = ((kb & np.int32(0xFFFF)) ^ 32768).astype(jnp.int16)
    lo16m = jnp.where(hi16 == p_hi16, lo16, np.int16(-32768))
    p_lo = jnp.zeros((1, _BLK), jnp.int32)
    for b in range(15, -1, -1):
        cand = p_lo | (1 << b)
        cand16 = (cand ^ 32768).astype(jnp.int16)
        cnt = _colcount(lo16m >= cand16)
        p_lo = jnp.where(cnt >= need, cand, p_lo)

    thr = ((p_hi << 16) | p_lo) ^ _INT_MIN
    maskf = jnp.where(ks >= thr, np.float32(1.0), np.float32(0.0))
    o_ref[...] = x_ref[...] * maskf.T


def kernel(x, x_emb, W):
    xe_t = x_emb.transpose(1, 2, 0).reshape(_FLAT, _B)
    wbf = W.astype(jnp.bfloat16)
    grid = (_B // _BLK,)
    return pl.pallas_call(
        _gate_kernel,
        grid=grid,
        in_specs=[
            pl.BlockSpec((_FLAT, _BLK), lambda i: (0, i)),
            pl.BlockSpec((_BLK, _BF), lambda i: (i, 0)),
            pl.BlockSpec((_BF, _NF), lambda i: (0, 0)),
        ],
        out_specs=pl.BlockSpec((_BLK, _BF), lambda i: (i, 0)),
        out_shape=jax.ShapeDtypeStruct((_B, _BF), jnp.float32),
        compiler_params=pltpu.CompilerParams(
            dimension_semantics=("arbitrary",),
        ),
    )(xe_t, x, wbf)


# speedup vs baseline: 9.3566x; 1.0014x over previous
import numpy as np
import jax
import jax.numpy as jnp
from jax.experimental import pallas as pl
from jax.experimental.pallas import tpu as pltpu

_K = 64
_TOPK_F = np.float32(_K)
_B = 16384
_NF = 100
_E = 16
_BF = 512
_FLAT = _NF * _E
_BLK = 2048

_INT_MIN = np.int32(-2147483648)
_POS_MASK = np.int32(0x7FFFFFFF)


def _gate_kernel(xe_ref, x_ref, w_ref, o_ref):
    xe = xe_ref[...]
    g = xe.reshape(_NF, _E, _BLK)
    ls = jnp.sum(g, axis=1) * np.float32(1.0 / _E)
    sT = jnp.dot(w_ref[...], ls.astype(jnp.bfloat16),
                 preferred_element_type=jnp.float32)
    sT = sT + np.float32(0.0)
    bits = jax.lax.bitcast_convert_type(sT, jnp.int32)
    ks = jnp.where(bits >= 0, bits, bits ^ _POS_MASK)
    kb = ks ^ _INT_MIN

    one16 = np.int16(1)
    zero16 = np.int16(0)

    def _colcount(m):
        v = jnp.where(m, one16, zero16)
        v = v[:256] + v[256:]
        v = v[:128] + v[128:]
        v = v[:64] + v[64:]
        v = v[:32] + v[32:]
        v = v[:16] + v[16:]
        return jnp.sum(v.astype(jnp.float32), axis=0, keepdims=True)

    hi16 = (jax.lax.shift_right_logical(kb, 16) ^ 32768).astype(jnp.int16)
    p_hi = jnp.zeros((1, _BLK), jnp.int32)
    for b in range(15, -1, -1):
        cand = p_hi | (1 << b)
        cand16 = (cand ^ 32768).astype(jnp.int16)
        cnt = _colcount(hi16 >= cand16)
        p_hi = jnp.where(cnt >= _TOPK_F, cand, p_hi)

    p_hi16 = (p_hi ^ 32768).astype(jnp.int16)
    c_gt = _colcount(hi16 > p_hi16)
    need = _TOPK_F - c_gt

    lo16 = ((kb & np.int32(0xFFFF)) ^ 32768).astype(jnp.int16)
    lo16m = jnp.where(hi16 == p_hi16, lo16, np.int16(-32768))
    p_lo = jnp.zeros((1, _BLK), jnp.int32)
    for b in range(15, -1, -1):
        cand = p_lo | (1 << b)
        cand16 = (cand ^ 32768).astype(jnp.int16)
        cnt = _colcount(lo16m >= cand16)
        p_lo = jnp.where(cnt >= need, cand, p_lo)

    thr = ((p_hi << 16) | p_lo) ^ _INT_MIN
    maskf = jnp.where(ks >= thr, np.float32(1.0), np.float32(0.0))
    o_ref[...] = x_ref[...] * maskf.T


def kernel(x, x_emb, W):
    xe_t = x_emb.transpose(1, 2, 0).reshape(_FLAT, _B)
    wbf = W.astype(jnp.bfloat16)
    grid = (_B // _BLK,)
    return pl.pallas_call(
        _gate_kernel,
        grid=grid,
        in_specs=[
            pl.BlockSpec((_FLAT, _BLK), lambda i: (0, i)),
            pl.BlockSpec((_BLK, _BF), lambda i: (i, 0)),
            pl.BlockSpec((_BF, _NF), lambda i: (0, 0)),
        ],
        out_specs=pl.BlockSpec((_BLK, _BF), lambda i: (i, 0)),
        out_shape=jax.ShapeDtypeStruct((_B, _BF), jnp.float32),
        compiler_params=pltpu.CompilerParams(
            dimension_semantics=("arbitrary",),
        ),
    )(xe_t, x, wbf)


# speedup vs baseline: 9.3629x; 1.0007x over previous
import numpy as np
import jax
import jax.numpy as jnp
from jax.experimental import pallas as pl
from jax.experimental.pallas import tpu as pltpu

_K = 64
_TOPK_F = np.float32(_K)
_B = 16384
_NF = 100
_E = 16
_BF = 512
_FLAT = _NF * _E
_BLK = 2048

_INT_MIN = np.int32(-2147483648)
_POS_MASK = np.int32(0x7FFFFFFF)


def _gate_kernel(xe_ref, x_ref, w_ref, o_ref):
    xe = xe_ref[...]
    g = xe.reshape(_NF, _E, _BLK)
    ls = jnp.sum(g, axis=1) * np.float32(1.0 / _E)
    sT = jnp.dot(w_ref[...], ls.astype(jnp.bfloat16),
                 preferred_element_type=jnp.float32)
    sT = sT + np.float32(0.0)
    bits = jax.lax.bitcast_convert_type(sT, jnp.int32)
    ks = jnp.where(bits >= 0, bits, bits ^ _POS_MASK)
    kb = ks ^ _INT_MIN

    one16 = np.int16(1)
    zero16 = np.int16(0)

    def _colcount(m):
        v = jnp.where(m, one16, zero16)
        v = v[:256] + v[256:]
        v = v[:128] + v[128:]
        v = v[:64] + v[64:]
        v = v[:32] + v[32:]
        v = v[:16] + v[16:]
        return jnp.sum(v.astype(jnp.float32), axis=0, keepdims=True)

    hi16 = (jax.lax.shift_right_logical(kb, 16) ^ 32768).astype(jnp.int16)
    p_hi = jnp.zeros((1, _BLK), jnp.int32)
    for b in range(15, -1, -1):
        cand = p_hi | (1 << b)
        cand16 = (cand ^ 32768).astype(jnp.int16)
        cnt = _colcount(hi16 >= cand16)
        p_hi = jnp.where(cnt >= _TOPK_F, cand, p_hi)

    p_hi16 = (p_hi ^ 32768).astype(jnp.int16)
    c_gt = _colcount(hi16 > p_hi16)
    need = _TOPK_F - c_gt

    lo16 = ((kb & np.int32(0xFFFF)) ^ 32768).astype(jnp.int16)
    lo16m = jnp.where(hi16 == p_hi16, lo16, np.int16(-32768))
    p_lo = jnp.zeros((1, _BLK), jnp.int32)
    for b in range(15, -1, -1):
        cand = p_lo | (1 << b)
        cand16 = (cand ^ 32768).astype(jnp.int16)
        cnt = _colcount(lo16m >= cand16)
        p_lo = jnp.where(cnt >= need, cand, p_lo)

    thr = ((p_hi << 16) | p_lo) ^ _INT_MIN
    maskf = jnp.where(ks >= thr, np.float32(1.0), np.float32(0.0))
    o_ref[...] = x_ref[...] * maskf.T


def kernel(x, x_emb, W):
    xe_t = x_emb.transpose(1, 2, 0).reshape(_FLAT, _B)
    wbf = W.astype(jnp.bfloat16)
    grid = (_B // _BLK,)
    return pl.pallas_call(
        _gate_kernel,
        grid=grid,
        in_specs=[
            pl.BlockSpec((_FLAT, _BLK), lambda i: (0, i)),
            pl.BlockSpec((_BLK, _BF), lambda i: (i, 0)),
            pl.BlockSpec((_BF, _NF), lambda i: (0, 0)),
        ],
        out_specs=pl.BlockSpec((_BLK, _BF), lambda i: (i, 0)),
        out_shape=jax.ShapeDtypeStruct((_B, _BF), jnp.float32),
        compiler_params=pltpu.CompilerParams(
            dimension_semantics=("parallel",),
        ),
    )(xe_t, x, wbf)
